# baseline (speedup 1.0000x reference)
import functools

import jax
import jax.numpy as jnp
from jax import lax
from jax.experimental import pallas as pl
from jax.experimental.pallas import tpu as pltpu
from jax.experimental.pallas import tpu_sc as plsc

_P = 7
_SCALE = 56.0
_L = 16


def _bin_bounds(rois, H, W):
    rs_w = jnp.round(rois[:, 1] * _SCALE).astype(jnp.int32)
    rs_h = jnp.round(rois[:, 2] * _SCALE).astype(jnp.int32)
    re_w = jnp.round(rois[:, 3] * _SCALE).astype(jnp.int32)
    re_h = jnp.round(rois[:, 4] * _SCALE).astype(jnp.int32)
    roi_w = jnp.maximum(re_w - rs_w + 1, 1)
    roi_h = jnp.maximum(re_h - rs_h + 1, 1)
    bin_h = roi_h.astype(jnp.float32) / _P
    bin_w = roi_w.astype(jnp.float32) / _P
    hs_l, he_l, ws_l, we_l = [], [], [], []
    for p in range(_P):
        hs_l.append(jnp.clip(jnp.floor(p * bin_h).astype(jnp.int32) + rs_h, 0, H))
        he_l.append(jnp.clip(jnp.ceil((p + 1) * bin_h).astype(jnp.int32) + rs_h, 0, H))
        ws_l.append(jnp.clip(jnp.floor(p * bin_w).astype(jnp.int32) + rs_w, 0, W))
        we_l.append(jnp.clip(jnp.ceil((p + 1) * bin_w).astype(jnp.int32) + rs_w, 0, W))
    hs = jnp.stack(hs_l, axis=1)
    he = jnp.stack(he_l, axis=1)
    ws = jnp.stack(ws_l, axis=1)
    we = jnp.stack(we_l, axis=1)
    return hs, he, ws, we


def _sc_roi_pool(feat_flat, bnd, R_pad, H, W, C):
    NC, NS = 2, 16
    NW = NC * NS
    RPW = R_pad // NW
    NCH = C // _L
    ROW = W * C
    OUTR = C * _P * _P
    mesh = plsc.VectorSubcoreMesh(core_axis_name="c", subcore_axis_name="s",
                                  num_cores=NC, num_subcores=NS)

    @functools.partial(
        pl.kernel,
        mesh=mesh,
        out_type=jax.ShapeDtypeStruct((R_pad * OUTR,), jnp.float32),
        scratch_types=[
            pltpu.VMEM((32,), jnp.int32),
            pltpu.VMEM((ROW,), jnp.float32),
            pltpu.VMEM((2 * ROW,), jnp.float32),
            pltpu.VMEM((OUTR,), jnp.float32),
            pltpu.SemaphoreType.DMA,
            pltpu.SemaphoreType.DMA,
            pltpu.SemaphoreType.DMA,
        ],
    )
    def k(feat_hbm, bnd_hbm, out_hbm, bnd_v, band_v, rows_v, acc_v,
          semz, sema, semb):
        wid = lax.axis_index("s") * NC + lax.axis_index("c")
        lane = lax.iota(jnp.int32, _L)
        ninf = jnp.full((_L,), -jnp.inf, jnp.float32)
        zero = jnp.zeros((_L,), jnp.float32)

        def extract(j):
            v = bnd_v[pl.ds((j // _L) * _L, _L)]
            return v[j % _L]

        def do_roi(i, _):
            r = wid * RPW + i
            pltpu.sync_copy(bnd_hbm.at[pl.ds(r * 32, 32)], bnd_v)
            w0 = extract(2 * _P)
            w1 = extract(4 * _P - 1)

            for ph in range(_P):
                hs = extract(ph)
                he = extract(_P + ph)
                nh = he - hs

                def wacc_from(off):
                    def wacc(w, _):
                        for c in range(NCH):
                            sl = pl.ds(w * C + c * _L, _L)
                            band_v[sl] = jnp.maximum(
                                band_v[sl],
                                rows_v[pl.ds(off + w * C + c * _L, _L)])
                        return 0
                    lax.fori_loop(w0, w1, wacc, 0)

                @pl.when(nh > 0)
                def _band():
                    cb = pltpu.async_copy(
                        feat_hbm.at[pl.ds(hs * ROW, ROW)], band_v, semz)

                    @pl.when(nh > 1)
                    def _p1():
                        pltpu.async_copy(
                            feat_hbm.at[pl.ds((hs + 1) * ROW, ROW)],
                            rows_v.at[pl.ds(0, ROW)], sema)

                    cb.wait()

                    def pair(kk, _):
                        d1 = 2 * kk + 1

                        @pl.when(d1 + 1 < nh)
                        def _pf_even():
                            pltpu.async_copy(
                                feat_hbm.at[pl.ds((hs + d1 + 1) * ROW, ROW)],
                                rows_v.at[pl.ds(ROW, ROW)], semb)

                        pltpu.make_async_copy(
                            feat_hbm.at[pl.ds((hs + d1) * ROW, ROW)],
                            rows_v.at[pl.ds(0, ROW)], sema).wait()
                        wacc_from(0)

                        @pl.when(d1 + 2 < nh)
                        def _pf_odd():
                            pltpu.async_copy(
                                feat_hbm.at[pl.ds((hs + d1 + 2) * ROW, ROW)],
                                rows_v.at[pl.ds(0, ROW)], sema)

                        @pl.when(d1 + 1 < nh)
                        def _even():
                            pltpu.make_async_copy(
                                feat_hbm.at[pl.ds((hs + d1 + 1) * ROW, ROW)],
                                rows_v.at[pl.ds(ROW, ROW)], semb).wait()
                            wacc_from(ROW)
                        return 0

                    lax.fori_loop(0, nh // 2, pair, 0)

                for pw in range(_P):
                    ws = extract(2 * _P + pw)
                    we = extract(3 * _P + pw)
                    obase = ph * _P + pw

                    def w_step(w, carry):
                        return tuple(
                            jnp.maximum(carry[c],
                                        band_v[pl.ds(w * C + c * _L, _L)])
                            for c in range(NCH))

                    mx = lax.fori_loop(ws, we, w_step,
                                       tuple(ninf for _ in range(NCH)))
                    @pl.when(nh > 0)
                    def _fill():
                        for c in range(NCH):
                            val = jnp.where(mx[c] > ninf, mx[c], zero)
                            acc_v[pl.ds(obase * C + c * _L, _L)] = val

                    @pl.when(nh == 0)
                    def _zero():
                        for c in range(NCH):
                            acc_v[pl.ds(obase * C + c * _L, _L)] = zero

            pltpu.sync_copy(acc_v, out_hbm.at[pl.ds(r * OUTR, OUTR)])
            return 0

        lax.fori_loop(0, RPW, do_roi, 0)

    return k(feat_flat, bnd)


_TRB = 8


def _tc_transpose(x):
    R, B, C = x.shape

    def body(x_ref, o_ref):
        o_ref[...] = jnp.swapaxes(x_ref[...], 1, 2)

    return pl.pallas_call(
        body,
        grid=(R // _TRB,),
        in_specs=[pl.BlockSpec((_TRB, B, C), lambda i: (i, 0, 0))],
        out_specs=pl.BlockSpec((_TRB, C, B), lambda i: (i, 0, 0)),
        out_shape=jax.ShapeDtypeStruct((R, C, B), jnp.float32),
    )(x)


@jax.jit
def kernel(input, rois):
    N, C, H, W = input.shape
    R = rois.shape[0]
    NW = 32
    R_pad = ((R + NW - 1) // NW) * NW
    feat = jnp.transpose(input[0], (1, 2, 0)).reshape(-1)
    hs, he, ws, we = _bin_bounds(rois, H, W)
    pad = jnp.zeros((R_pad - R, _P), jnp.int32)
    bnd = jnp.concatenate([
        jnp.concatenate([hs, pad], 0),
        jnp.concatenate([he, pad], 0),
        jnp.concatenate([ws, pad], 0),
        jnp.concatenate([we, pad], 0),
        jnp.zeros((R_pad, 4), jnp.int32),
    ], axis=1).reshape(-1)
    out = _sc_roi_pool(feat, bnd, R_pad, H, W, C)
    out = out.reshape(R_pad, _P * _P, C)[:R]
    out = _tc_transpose(out)
    return out.reshape(R, C, _P, _P)


# speedup vs baseline: 1.2796x; 1.2796x over previous
import functools

import jax
import jax.numpy as jnp
from jax import lax
from jax.experimental import pallas as pl
from jax.experimental.pallas import tpu as pltpu
from jax.experimental.pallas import tpu_sc as plsc

_P = 7
_SCALE = 56.0
_L = 16


def _bin_bounds(rois, H, W):
    rs_w = jnp.round(rois[:, 1] * _SCALE).astype(jnp.int32)
    rs_h = jnp.round(rois[:, 2] * _SCALE).astype(jnp.int32)
    re_w = jnp.round(rois[:, 3] * _SCALE).astype(jnp.int32)
    re_h = jnp.round(rois[:, 4] * _SCALE).astype(jnp.int32)
    roi_w = jnp.maximum(re_w - rs_w + 1, 1)
    roi_h = jnp.maximum(re_h - rs_h + 1, 1)
    bin_h = roi_h.astype(jnp.float32) / _P
    bin_w = roi_w.astype(jnp.float32) / _P
    hs_l, he_l, ws_l, we_l = [], [], [], []
    for p in range(_P):
        hs_l.append(jnp.clip(jnp.floor(p * bin_h).astype(jnp.int32) + rs_h, 0, H))
        he_l.append(jnp.clip(jnp.ceil((p + 1) * bin_h).astype(jnp.int32) + rs_h, 0, H))
        ws_l.append(jnp.clip(jnp.floor(p * bin_w).astype(jnp.int32) + rs_w, 0, W))
        we_l.append(jnp.clip(jnp.ceil((p + 1) * bin_w).astype(jnp.int32) + rs_w, 0, W))
    hs = jnp.stack(hs_l, axis=1)
    he = jnp.stack(he_l, axis=1)
    ws = jnp.stack(ws_l, axis=1)
    we = jnp.stack(we_l, axis=1)
    return hs, he, ws, we


def _sc_roi_pool(feat_flat, bnd, R_pad, H, W, C):
    NC, NS = 2, 16
    NW = NC * NS
    RPW = R_pad // NW
    NCH = C // _L
    ROW = W * C
    OUTR = C * _P * _P
    mesh = plsc.VectorSubcoreMesh(core_axis_name="c", subcore_axis_name="s",
                                  num_cores=NC, num_subcores=NS)

    @functools.partial(
        pl.kernel,
        mesh=mesh,
        out_type=jax.ShapeDtypeStruct((R_pad * OUTR,), jnp.float32),
        scratch_types=[
            pltpu.VMEM((32,), jnp.int32),
            pltpu.VMEM((ROW,), jnp.float32),
            pltpu.VMEM((2 * ROW,), jnp.float32),
            pltpu.VMEM((OUTR,), jnp.float32),
            pltpu.SemaphoreType.DMA,
            pltpu.SemaphoreType.DMA,
            pltpu.SemaphoreType.DMA,
        ],
    )
    def k(feat_hbm, bnd_hbm, out_hbm, bnd_v, band_v, rows_v, acc_v,
          semz, sema, semb):
        wid = lax.axis_index("s") * NC + lax.axis_index("c")
        lane = lax.iota(jnp.int32, _L)
        ninf = jnp.full((_L,), -jnp.inf, jnp.float32)
        zero = jnp.zeros((_L,), jnp.float32)

        def extract(j):
            v = bnd_v[pl.ds((j // _L) * _L, _L)]
            return v[j % _L]

        def do_roi(i, _):
            r = wid * RPW + i
            pltpu.sync_copy(bnd_hbm.at[pl.ds(r * 32, 32)], bnd_v)
            w0 = extract(2 * _P)
            w1 = extract(4 * _P - 1)

            wa8 = (w0 // 8) * 8
            ncw = (w1 - wa8 + 7) // 8

            def row_issue(h, ref, base, sem):
                def cdma(t, _):
                    wo = wa8 + t * 8
                    pltpu.async_copy(
                        feat_hbm.at[pl.ds((h * W + wo) * C, 8 * C)],
                        ref.at[pl.ds(base + wo * C, 8 * C)], sem)
                    return 0
                lax.fori_loop(0, ncw, cdma, 0)

            def row_wait(h, ref, base, sem):
                def cw(t, _):
                    wo = wa8 + t * 8
                    pltpu.make_async_copy(
                        feat_hbm.at[pl.ds((h * W + wo) * C, 8 * C)],
                        ref.at[pl.ds(base + wo * C, 8 * C)], sem).wait()
                    return 0
                lax.fori_loop(0, ncw, cw, 0)

            for ph in range(_P):
                hs = extract(ph)
                he = extract(_P + ph)
                nh = he - hs

                def wacc_from(off):
                    def wacc(w, _):
                        for c in range(NCH):
                            sl = pl.ds(w * C + c * _L, _L)
                            band_v[sl] = jnp.maximum(
                                band_v[sl],
                                rows_v[pl.ds(off + w * C + c * _L, _L)])
                        return 0
                    lax.fori_loop(w0, w1, wacc, 0)

                @pl.when(nh > 0)
                def _band():
                    row_issue(hs, band_v, 0, semz)

                    @pl.when(nh > 1)
                    def _p1():
                        row_issue(hs + 1, rows_v, 0, sema)

                    row_wait(hs, band_v, 0, semz)

                    def pair(kk, _):
                        d1 = 2 * kk + 1

                        @pl.when(d1 + 1 < nh)
                        def _pf_even():
                            row_issue(hs + d1 + 1, rows_v, ROW, semb)

                        row_wait(hs + d1, rows_v, 0, sema)
                        wacc_from(0)

                        @pl.when(d1 + 2 < nh)
                        def _pf_odd():
                            row_issue(hs + d1 + 2, rows_v, 0, sema)

                        @pl.when(d1 + 1 < nh)
                        def _even():
                            row_wait(hs + d1 + 1, rows_v, ROW, semb)
                            wacc_from(ROW)
                        return 0

                    lax.fori_loop(0, nh // 2, pair, 0)

                for pw in range(_P):
                    ws = extract(2 * _P + pw)
                    we = extract(3 * _P + pw)
                    obase = ph * _P + pw

                    def w_step(w, carry):
                        return tuple(
                            jnp.maximum(carry[c],
                                        band_v[pl.ds(w * C + c * _L, _L)])
                            for c in range(NCH))

                    mx = lax.fori_loop(ws, we, w_step,
                                       tuple(ninf for _ in range(NCH)))
                    @pl.when(nh > 0)
                    def _fill():
                        for c in range(NCH):
                            val = jnp.where(mx[c] > ninf, mx[c], zero)
                            acc_v[pl.ds(obase * C + c * _L, _L)] = val

                    @pl.when(nh == 0)
                    def _zero():
                        for c in range(NCH):
                            acc_v[pl.ds(obase * C + c * _L, _L)] = zero

            pltpu.sync_copy(acc_v, out_hbm.at[pl.ds(r * OUTR, OUTR)])
            return 0

        lax.fori_loop(0, RPW, do_roi, 0)

    return k(feat_flat, bnd)


_TRB = 8


def _tc_transpose(x):
    R, B, C = x.shape

    def body(x_ref, o_ref):
        o_ref[...] = jnp.swapaxes(x_ref[...], 1, 2)

    return pl.pallas_call(
        body,
        grid=(R // _TRB,),
        in_specs=[pl.BlockSpec((_TRB, B, C), lambda i: (i, 0, 0))],
        out_specs=pl.BlockSpec((_TRB, C, B), lambda i: (i, 0, 0)),
        out_shape=jax.ShapeDtypeStruct((R, C, B), jnp.float32),
    )(x)


@jax.jit
def kernel(input, rois):
    N, C, H, W = input.shape
    R = rois.shape[0]
    NW = 32
    R_pad = ((R + NW - 1) // NW) * NW
    feat = jnp.transpose(input[0], (1, 2, 0)).reshape(-1)
    hs, he, ws, we = _bin_bounds(rois, H, W)
    pad = jnp.zeros((R_pad - R, _P), jnp.int32)
    bnd = jnp.concatenate([
        jnp.concatenate([hs, pad], 0),
        jnp.concatenate([he, pad], 0),
        jnp.concatenate([ws, pad], 0),
        jnp.concatenate([we, pad], 0),
        jnp.zeros((R_pad, 4), jnp.int32),
    ], axis=1).reshape(-1)
    out = _sc_roi_pool(feat, bnd, R_pad, H, W, C)
    out = out.reshape(R_pad, _P * _P, C)[:R]
    return jnp.transpose(out, (0, 2, 1)).reshape(R, C, _P, _P)
